# baseline (speedup 1.0000x reference)
import functools

import jax
import jax.numpy as jnp
from jax import lax
from jax.experimental import pallas as pl
from jax.experimental.pallas import tpu as pltpu
from jax.experimental.pallas import tpu_sc as plsc

N_TOK = 36864
N_CW = 8192
D = 64

BN = 1024
KC = 512
NKC = N_CW // KC

_NC = 2
_NS = 16
_NW = _NC * _NS
_BPW = N_TOK // _NW
_CH = 128
_NCH = _BPW // _CH

_BIG = float(3.0e38)


def _argmax_body(x_ref, x2_ref, cbt_ref, c2_ref, iota_ref, out_ref):
    x2x = x_ref[...] * 2.0
    x2b = x2_ref[...]
    ms, pks = [], []
    for c in range(NKC):
        sl = pl.ds(c * KC, KC)
        mm2 = lax.dot_general(
            x2x, cbt_ref[:, sl], (((1,), (0,)), ((), ())),
            preferred_element_type=jnp.float32)
        d = (x2b + c2_ref[:, sl]) - mm2
        m_c = jnp.min(d, axis=1, keepdims=True)
        pk_c = jnp.min(jnp.where(d == m_c, iota_ref[:, sl], _BIG),
                       axis=1, keepdims=True)
        ms.append(m_c)
        pks.append(pk_c)
    msc = jnp.concatenate(ms, axis=1)
    pksc = jnp.concatenate(pks, axis=1)
    m = jnp.min(msc, axis=1, keepdims=True)
    idx = jnp.min(jnp.where(msc == m, pksc, _BIG), axis=1)
    out_ref[...] = idx.astype(jnp.int32)[:, None]


_argmax_call = pl.pallas_call(
    _argmax_body,
    grid=(N_TOK // BN,),
    in_specs=[
        pl.BlockSpec((BN, D), lambda i: (i, 0)),
        pl.BlockSpec((BN, 1), lambda i: (i, 0)),
        pl.BlockSpec((D, N_CW), lambda i: (0, 0)),
        pl.BlockSpec((1, N_CW), lambda i: (0, 0)),
        pl.BlockSpec((1, N_CW), lambda i: (0, 0)),
    ],
    out_specs=pl.BlockSpec((BN, 1), lambda i: (i, 0)),
    out_shape=jax.ShapeDtypeStruct((N_TOK, 1), jnp.int32),
)


def _gather_body(cb_hbm, idx_hbm, out_hbm, idx_v, rows_v, sem):
    wid = lax.axis_index("s") * _NC + lax.axis_index("c")
    pltpu.sync_copy(idx_hbm.at[pl.ds(wid * _BPW, _BPW)], idx_v)
    copies = [
        pltpu.async_copy(cb_hbm.at[idx_v.at[pl.ds(j * _CH, _CH)]],
                         rows_v.at[pl.ds(j * _CH, _CH)], sem)
        for j in range(_NCH)
    ]
    for cp in copies:
        cp.wait()
    pltpu.sync_copy(rows_v, out_hbm.at[pl.ds(wid * _BPW, _BPW)])


def _make_gather_call():
    return functools.partial(
        pl.kernel,
        out_type=jax.ShapeDtypeStruct((N_TOK, D), jnp.float32),
        scratch_types=[
            pltpu.VMEM((_BPW,), jnp.int32),
            pltpu.VMEM((_BPW, D), jnp.float32),
            pltpu.SemaphoreType.DMA,
        ],
        mesh=plsc.VectorSubcoreMesh(core_axis_name="c",
                                    subcore_axis_name="s"),
        compiler_params=pltpu.CompilerParams(use_tc_tiling_on_sc=False),
    )(_gather_body)


def kernel(x, CodeBook):
    x2 = jnp.sum(x * x, axis=1, keepdims=True)
    c2 = jnp.sum(CodeBook * CodeBook, axis=1)[None, :]
    cbt = CodeBook.T
    iota = lax.broadcasted_iota(jnp.float32, (1, N_CW), 1)
    max_id = _argmax_call(x, x2, cbt, c2, iota).reshape(N_TOK)
    q_hard = _make_gather_call()(CodeBook, max_id)
    return (q_hard, max_id)


# speedup vs baseline: 1.0231x; 1.0231x over previous
import functools

import jax
import jax.numpy as jnp
from jax import lax
from jax.experimental import pallas as pl
from jax.experimental.pallas import tpu as pltpu
from jax.experimental.pallas import tpu_sc as plsc

N_TOK = 36864
N_CW = 8192
D = 64

BN = 512
KC = 1024
NKC = N_CW // KC

_NC = 2
_NS = 16
_NW = _NC * _NS
_BPW = N_TOK // _NW
_CH = 128
_NCH = _BPW // _CH

_BIG = float(3.0e38)


def _argmax_body(x_ref, x2_ref, cbt_ref, c2_ref, iota_ref, out_ref):
    x2x = x_ref[...] * 2.0
    x2b = x2_ref[...]
    ms, pks = [], []
    for c in range(NKC):
        sl = pl.ds(c * KC, KC)
        mm2 = lax.dot_general(
            x2x, cbt_ref[:, sl], (((1,), (0,)), ((), ())),
            preferred_element_type=jnp.float32)
        d = (x2b + c2_ref[:, sl]) - mm2
        m_c = jnp.min(d, axis=1, keepdims=True)
        pk_c = jnp.min(jnp.where(d == m_c, iota_ref[:, sl], _BIG),
                       axis=1, keepdims=True)
        ms.append(m_c)
        pks.append(pk_c)
    msc = jnp.concatenate(ms, axis=1)
    pksc = jnp.concatenate(pks, axis=1)
    m = jnp.min(msc, axis=1, keepdims=True)
    idx = jnp.min(jnp.where(msc == m, pksc, _BIG), axis=1)
    out_ref[...] = idx.astype(jnp.int32)[:, None]


_argmax_call = pl.pallas_call(
    _argmax_body,
    grid=(N_TOK // BN,),
    in_specs=[
        pl.BlockSpec((BN, D), lambda i: (i, 0)),
        pl.BlockSpec((BN, 1), lambda i: (i, 0)),
        pl.BlockSpec((D, N_CW), lambda i: (0, 0)),
        pl.BlockSpec((1, N_CW), lambda i: (0, 0)),
        pl.BlockSpec((1, N_CW), lambda i: (0, 0)),
    ],
    out_specs=pl.BlockSpec((BN, 1), lambda i: (i, 0)),
    out_shape=jax.ShapeDtypeStruct((N_TOK, 1), jnp.int32),
)


def _gather_body(cb_hbm, idx_hbm, out_hbm, idx_v, rows_v, sem):
    wid = lax.axis_index("s") * _NC + lax.axis_index("c")
    pltpu.sync_copy(idx_hbm.at[pl.ds(wid * _BPW, _BPW)], idx_v)
    copies = [
        pltpu.async_copy(cb_hbm.at[idx_v.at[pl.ds(j * _CH, _CH)]],
                         rows_v.at[pl.ds(j * _CH, _CH)], sem)
        for j in range(_NCH)
    ]
    for cp in copies:
        cp.wait()
    pltpu.sync_copy(rows_v, out_hbm.at[pl.ds(wid * _BPW, _BPW)])


def _make_gather_call():
    return functools.partial(
        pl.kernel,
        out_type=jax.ShapeDtypeStruct((N_TOK, D), jnp.float32),
        scratch_types=[
            pltpu.VMEM((_BPW,), jnp.int32),
            pltpu.VMEM((_BPW, D), jnp.float32),
            pltpu.SemaphoreType.DMA,
        ],
        mesh=plsc.VectorSubcoreMesh(core_axis_name="c",
                                    subcore_axis_name="s"),
        compiler_params=pltpu.CompilerParams(use_tc_tiling_on_sc=False),
    )(_gather_body)


def kernel(x, CodeBook):
    x2 = jnp.sum(x * x, axis=1, keepdims=True)
    c2 = jnp.sum(CodeBook * CodeBook, axis=1)[None, :]
    cbt = CodeBook.T
    iota = lax.broadcasted_iota(jnp.float32, (1, N_CW), 1)
    max_id = _argmax_call(x, x2, cbt, c2, iota).reshape(N_TOK)
    q_hard = _make_gather_call()(CodeBook, max_id)
    return (q_hard, max_id)


# speedup vs baseline: 1.0653x; 1.0412x over previous
import functools

import jax
import jax.numpy as jnp
from jax import lax
from jax.experimental import pallas as pl
from jax.experimental.pallas import tpu as pltpu
from jax.experimental.pallas import tpu_sc as plsc

N_TOK = 36864
N_CW = 8192
D = 64

BN = 2048
KC = 1024
NKC = N_CW // KC

_NC = 2
_NS = 16
_NW = _NC * _NS
_BPW = N_TOK // _NW
_CH = 128
_NCH = _BPW // _CH

_BIG = float(3.0e38)


def _argmax_body(x_ref, x2_ref, cbt_ref, c2_ref, iota_ref, out_ref):
    x2x = x_ref[...] * 2.0
    x2b = x2_ref[...]
    ms, pks = [], []
    for c in range(NKC):
        sl = pl.ds(c * KC, KC)
        mm2 = lax.dot_general(
            x2x, cbt_ref[:, sl], (((1,), (0,)), ((), ())),
            preferred_element_type=jnp.float32)
        d = (x2b + c2_ref[:, sl]) - mm2
        m_c = jnp.min(d, axis=1, keepdims=True)
        pk_c = jnp.min(jnp.where(d == m_c, iota_ref[:, sl], _BIG),
                       axis=1, keepdims=True)
        ms.append(m_c)
        pks.append(pk_c)
    msc = jnp.concatenate(ms, axis=1)
    pksc = jnp.concatenate(pks, axis=1)
    m = jnp.min(msc, axis=1, keepdims=True)
    idx = jnp.min(jnp.where(msc == m, pksc, _BIG), axis=1)
    out_ref[...] = idx.astype(jnp.int32)[:, None]


_argmax_call = pl.pallas_call(
    _argmax_body,
    grid=(N_TOK // BN,),
    in_specs=[
        pl.BlockSpec((BN, D), lambda i: (i, 0)),
        pl.BlockSpec((BN, 1), lambda i: (i, 0)),
        pl.BlockSpec((D, N_CW), lambda i: (0, 0)),
        pl.BlockSpec((1, N_CW), lambda i: (0, 0)),
        pl.BlockSpec((1, N_CW), lambda i: (0, 0)),
    ],
    out_specs=pl.BlockSpec((BN, 1), lambda i: (i, 0)),
    out_shape=jax.ShapeDtypeStruct((N_TOK, 1), jnp.int32),
)


def _gather_body(cb_hbm, idx_hbm, out_hbm, idx_v, rows_v, sem):
    wid = lax.axis_index("s") * _NC + lax.axis_index("c")
    pltpu.sync_copy(idx_hbm.at[pl.ds(wid * _BPW, _BPW)], idx_v)
    copies = [
        pltpu.async_copy(cb_hbm.at[idx_v.at[pl.ds(j * _CH, _CH)]],
                         rows_v.at[pl.ds(j * _CH, _CH)], sem)
        for j in range(_NCH)
    ]
    for cp in copies:
        cp.wait()
    pltpu.sync_copy(rows_v, out_hbm.at[pl.ds(wid * _BPW, _BPW)])


def _make_gather_call():
    return functools.partial(
        pl.kernel,
        out_type=jax.ShapeDtypeStruct((N_TOK, D), jnp.float32),
        scratch_types=[
            pltpu.VMEM((_BPW,), jnp.int32),
            pltpu.VMEM((_BPW, D), jnp.float32),
            pltpu.SemaphoreType.DMA,
        ],
        mesh=plsc.VectorSubcoreMesh(core_axis_name="c",
                                    subcore_axis_name="s"),
        compiler_params=pltpu.CompilerParams(use_tc_tiling_on_sc=False),
    )(_gather_body)


def kernel(x, CodeBook):
    x2 = jnp.sum(x * x, axis=1, keepdims=True)
    c2 = jnp.sum(CodeBook * CodeBook, axis=1)[None, :]
    cbt = CodeBook.T
    iota = lax.broadcasted_iota(jnp.float32, (1, N_CW), 1)
    max_id = _argmax_call(x, x2, cbt, c2, iota).reshape(N_TOK)
    q_hard = _make_gather_call()(CodeBook, max_id)
    return (q_hard, max_id)


# speedup vs baseline: 1.1128x; 1.0445x over previous
import functools

import jax
import jax.numpy as jnp
from jax import lax
from jax.experimental import pallas as pl
from jax.experimental.pallas import tpu as pltpu
from jax.experimental.pallas import tpu_sc as plsc

N_TOK = 36864
N_CW = 8192
D = 64

BN = 2048
KC = 1024
NKC = N_CW // KC

_NC = 2
_NS = 16
_NW = _NC * _NS
_BPW = N_TOK // _NW
_CH = 128
_NCH = _BPW // _CH

_BIG = float(3.0e38)


def _argmax_body(x_ref, cb_ref, out_ref, cbt_s, c2_s):
    @pl.when(pl.program_id(0) == 0)
    def _init():
        cbt = jnp.transpose(cb_ref[...], (1, 0))
        cbt_s[...] = cbt
        c2_s[...] = jnp.sum(cbt * cbt, axis=0, keepdims=True)

    xb = x_ref[...]
    x2b = jnp.sum(xb * xb, axis=1, keepdims=True)
    x2x = xb * 2.0
    ms, pks = [], []
    for c in range(NKC):
        sl = pl.ds(c * KC, KC)
        mm2 = lax.dot_general(
            x2x, cbt_s[:, sl], (((1,), (0,)), ((), ())),
            preferred_element_type=jnp.float32)
        d = (x2b + c2_s[:, sl]) - mm2
        m_c = jnp.min(d, axis=1, keepdims=True)
        iota_c = (lax.broadcasted_iota(jnp.int32, (1, KC), 1)
                  + c * KC).astype(jnp.float32)
        pk_c = jnp.min(jnp.where(d == m_c, iota_c, _BIG),
                       axis=1, keepdims=True)
        ms.append(m_c)
        pks.append(pk_c)
    msc = jnp.concatenate(ms, axis=1)
    pksc = jnp.concatenate(pks, axis=1)
    m = jnp.min(msc, axis=1, keepdims=True)
    idx = jnp.min(jnp.where(msc == m, pksc, _BIG), axis=1)
    out_ref[...] = idx.astype(jnp.int32)[:, None]


_argmax_call = pl.pallas_call(
    _argmax_body,
    grid=(N_TOK // BN,),
    in_specs=[
        pl.BlockSpec((BN, D), lambda i: (i, 0)),
        pl.BlockSpec((N_CW, D), lambda i: (0, 0)),
    ],
    out_specs=pl.BlockSpec((BN, 1), lambda i: (i, 0)),
    out_shape=jax.ShapeDtypeStruct((N_TOK, 1), jnp.int32),
    scratch_shapes=[
        pltpu.VMEM((D, N_CW), jnp.float32),
        pltpu.VMEM((1, N_CW), jnp.float32),
    ],
)


def _gather_body(cb_hbm, idx_hbm, out_hbm, idx_v, rows_v, sem):
    wid = lax.axis_index("s") * _NC + lax.axis_index("c")
    pltpu.sync_copy(idx_hbm.at[pl.ds(wid * _BPW, _BPW)], idx_v)
    copies = [
        pltpu.async_copy(cb_hbm.at[idx_v.at[pl.ds(j * _CH, _CH)]],
                         rows_v.at[pl.ds(j * _CH, _CH)], sem)
        for j in range(_NCH)
    ]
    for cp in copies:
        cp.wait()
    pltpu.sync_copy(rows_v, out_hbm.at[pl.ds(wid * _BPW, _BPW)])


def _make_gather_call():
    return functools.partial(
        pl.kernel,
        out_type=jax.ShapeDtypeStruct((N_TOK, D), jnp.float32),
        scratch_types=[
            pltpu.VMEM((_BPW,), jnp.int32),
            pltpu.VMEM((_BPW, D), jnp.float32),
            pltpu.SemaphoreType.DMA,
        ],
        mesh=plsc.VectorSubcoreMesh(core_axis_name="c",
                                    subcore_axis_name="s"),
        compiler_params=pltpu.CompilerParams(use_tc_tiling_on_sc=False),
    )(_gather_body)


def kernel(x, CodeBook):
    max_id = _argmax_call(x, CodeBook).reshape(N_TOK)
    q_hard = _make_gather_call()(CodeBook, max_id)
    return (q_hard, max_id)
